# baseline (speedup 1.0000x reference)
import functools

import jax
import jax.numpy as jnp
from jax import lax
from jax.experimental import pallas as pl
from jax.experimental.pallas import tpu as pltpu
from jax.experimental.pallas import tpu_sc as plsc

_EPS = 1e-5
_KNN = 9
_CPAD = 256
_HI = jax.lax.Precision.HIGHEST
_INV_SQRT2 = 0.7071067811865476


def _dot(a, b, dims, precision=_HI):
    return jax.lax.dot_general(a, b, (dims, ((), ())),
                               preferred_element_type=jnp.float32,
                               precision=precision)


def _accum_stats(s_ref, val, is_first):
    st = jnp.concatenate([jnp.sum(val, axis=0, keepdims=True),
                          jnp.sum(val * val, axis=0, keepdims=True)], axis=0)

    @pl.when(is_first)
    def _():
        s_ref[...] = jnp.zeros_like(s_ref)

    s_ref[...] += st


def _affine_from_stats(s_ref, g_ref, be_ref, n_tot):
    inv = 1.0 / n_tot
    mean = s_ref[0:1, :] * inv
    var = s_ref[1:2, :] * inv - mean * mean
    a = g_ref[...] * jax.lax.rsqrt(var + _EPS)
    d = be_ref[...] - mean * a
    return a, d


def _conv1_body(xt_ref, w1_ref, b1_ref, h_ref, s_ref):
    b = pl.program_id(0)
    h = _dot(xt_ref[0], w1_ref[...], ((1,), (1,)), precision=None) + b1_ref[...]
    h_ref[0] = h
    _accum_stats(s_ref, h, b == 0)


def _knn_body(n_tot, h_ref, s1_ref, g1_ref, be1_ref, xr_ref, gidx_ref):
    b = pl.program_id(0)
    a, d = _affine_from_stats(s1_ref, g1_ref, be1_ref, n_tot)
    xr = h_ref[0] * a + d
    xr_ref[0] = jnp.concatenate(
        [xr, jnp.zeros((xr.shape[0], _CPAD - xr.shape[1]), jnp.float32)],
        axis=1)
    nsq = jnp.sum(xr * xr, axis=1, keepdims=True)
    xn = xr * (1.0 / jnp.maximum(jnp.sqrt(nsq), 1e-12))
    n = xn.shape[0]
    sim = _dot(xn, xn, ((1,), (1,)), precision=None)
    sq_row = _dot(jnp.ones((8, xn.shape[1]), jnp.float32), xn * xn,
                  ((1,), (1,)))[0:1]
    v = sq_row - 2.0 * sim
    col = jax.lax.broadcasted_iota(jnp.int32, v.shape, 1)
    idxs = []
    for k in range(_KNN):
        rowmin = jnp.min(v, axis=1, keepdims=True)
        idx = jnp.min(jnp.where(v == rowmin, col, n), axis=1, keepdims=True)
        idxs.append(idx + b * n)
        v = jnp.where(col == idx, jnp.inf, v)
    gidx_ref[0] = jnp.concatenate(idxs, axis=1)


def _zgp_body(xr_ref, w_ref, zgp_ref):
    zgp_ref[0] = _dot(xr_ref[0], w_ref[...], ((1,), (0,)))


def _zg_body(zgp_ref, amax_ref, wdf_ref, bg_ref, zg_ref, s2_ref):
    b = pl.program_id(0)
    zg = zgp_ref[0] + _dot(amax_ref[0], wdf_ref[...], ((1,), (0,))) \
        + bg_ref[...]
    zg_ref[0] = zg
    _accum_stats(s2_ref, zg, b == 0)


_SC_ITEMS = 8
_SC_CHUNKS = 32


def _sc_gather_max(table_ref, idx_ref, out_ref, idxc, rows, outv, sem):
    wid = lax.axis_index("s") * 2 + lax.axis_index("c")
    base = wid * (_SC_ITEMS * _SC_CHUNKS)

    def chunk_body(ci, carry):
        start = base + ci * _SC_ITEMS
        pltpu.sync_copy(idx_ref.at[pl.ds(start * _KNN, _SC_ITEMS * _KNN)],
                        idxc)
        pltpu.async_copy(table_ref.at[idxc], rows, sem).wait()
        for i in range(_SC_ITEMS):
            for c in range(_CPAD // 16):
                sl = slice(c * 16, c * 16 + 16)
                m = rows[i * _KNN, sl]
                for k in range(1, _KNN):
                    m = jnp.maximum(m, rows[i * _KNN + k, sl])
                outv[i, sl] = m
        pltpu.sync_copy(outv, out_ref.at[pl.ds(start, _SC_ITEMS)])
        return carry

    lax.fori_loop(0, _SC_CHUNKS, chunk_body, 0)


def _head_body(n_tot, zg_ref, s2_ref, gg_ref, beg_ref, w2_ref, b2_ref,
               o_ref, s3_ref):
    b = pl.program_id(0)
    a, d = _affine_from_stats(s2_ref, gg_ref, beg_ref, n_tot)
    t = zg_ref[0] * a + d
    z = 0.5 * t * (1.0 + jax.lax.erf(t * _INV_SQRT2))
    o = _dot(z, w2_ref[...], ((1,), (1,))) + b2_ref[...]
    o_ref[0] = o
    _accum_stats(s3_ref, o, b == 0)


def _res_body(n_tot, o_ref, s3_ref, g2_ref, be2_ref, xt_ref, out_ref):
    a, d = _affine_from_stats(s3_ref, g2_ref, be2_ref, n_tot)
    out_ref[0] = o_ref[0] * a + d + xt_ref[0]


def _block_diag(wb, c_in, c_out, groups, dtype):
    ig, og = c_in // groups, c_out // groups
    m = jnp.zeros((c_in, c_out), dtype)
    for g in range(groups):
        m = m.at[g * ig:(g + 1) * ig, g * og:(g + 1) * og].set(
            jnp.transpose(wb[g]))
    return m


def kernel(x, W1, b1, g1, be1, Wg, bg, gg, beg, W2, b2, g2, be2):
    B, C, H, W = x.shape
    N = H * W
    Cout = Wg.shape[0]
    groups = 4
    n_tot = float(B * N)
    f32 = jnp.float32

    xt = jnp.transpose(x.reshape(B, C, N), (0, 2, 1))
    wg3 = Wg.reshape(groups, Cout // groups, (2 * C) // groups)
    wxr = _block_diag(wg3[:, :, 0::2], C, Cout, groups, f32)
    wdf = _block_diag(wg3[:, :, 1::2], C, Cout, groups, f32)

    row = lambda v: v.reshape(1, -1).astype(f32)
    b1r, g1r, be1r = row(b1), row(g1), row(be1)
    bgr, ggr, begr = row(bg), row(gg), row(beg)
    b2r, g2r, be2r = row(b2), row(g2), row(be2)

    full = lambda shape: pl.BlockSpec(shape, lambda b: (0,) * len(shape))
    per_b = lambda shape: pl.BlockSpec((1,) + shape,
                                       lambda b: (b,) + (0,) * len(shape))

    h_raw, s1 = pl.pallas_call(
        _conv1_body,
        grid=(B,),
        in_specs=[per_b((N, C)), full((C, C)), full((1, C))],
        out_specs=[per_b((N, C)), full((2, C))],
        out_shape=[jax.ShapeDtypeStruct((B, N, C), f32),
                   jax.ShapeDtypeStruct((2, C), f32)],
    )(xt, W1, b1r)

    xr_all, gidx = pl.pallas_call(
        functools.partial(_knn_body, n_tot),
        grid=(B,),
        in_specs=[per_b((N, C)), full((2, C)), full((1, C)), full((1, C))],
        out_specs=[per_b((N, _CPAD)), per_b((N, _KNN))],
        out_shape=[jax.ShapeDtypeStruct((B, N, _CPAD), f32),
                   jax.ShapeDtypeStruct((B, N, _KNN), jnp.int32)],
    )(h_raw, s1, g1r, be1r)

    sc_mesh = plsc.VectorSubcoreMesh(core_axis_name="c", subcore_axis_name="s")
    amax = pl.kernel(
        _sc_gather_max,
        mesh=sc_mesh,
        out_type=jax.ShapeDtypeStruct((B * N, _CPAD), f32),
        scratch_types=[
            pltpu.VMEM((_SC_ITEMS * _KNN,), jnp.int32),
            pltpu.VMEM((_SC_ITEMS * _KNN, _CPAD), f32),
            pltpu.VMEM((_SC_ITEMS, _CPAD), f32),
            pltpu.SemaphoreType.DMA,
        ],
    )(xr_all.reshape(B * N, _CPAD), gidx.reshape(B * N * _KNN))
    amax = amax.reshape(B, N, _CPAD)

    pad_rows = lambda w: jnp.concatenate(
        [w, jnp.zeros((_CPAD - C, Cout), f32)], axis=0)

    zgp = pl.pallas_call(
        _zgp_body,
        grid=(B,),
        in_specs=[per_b((N, _CPAD)), full((_CPAD, Cout))],
        out_specs=per_b((N, Cout)),
        out_shape=jax.ShapeDtypeStruct((B, N, Cout), f32),
    )(xr_all, pad_rows(wxr - wdf))

    zg_raw, s2 = pl.pallas_call(
        _zg_body,
        grid=(B,),
        in_specs=[per_b((N, Cout)), per_b((N, _CPAD)), full((_CPAD, Cout)),
                  full((1, Cout))],
        out_specs=[per_b((N, Cout)), full((2, Cout))],
        out_shape=[jax.ShapeDtypeStruct((B, N, Cout), f32),
                   jax.ShapeDtypeStruct((2, Cout), f32)],
    )(zgp, amax, pad_rows(wdf), bgr)

    o_raw, s3 = pl.pallas_call(
        functools.partial(_head_body, n_tot),
        grid=(B,),
        in_specs=[per_b((N, Cout)), full((2, Cout)), full((1, Cout)),
                  full((1, Cout)), full((C, Cout)), full((1, C))],
        out_specs=[per_b((N, C)), full((2, C))],
        out_shape=[jax.ShapeDtypeStruct((B, N, C), f32),
                   jax.ShapeDtypeStruct((2, C), f32)],
    )(zg_raw, s2, ggr, begr, W2, b2r)

    out = pl.pallas_call(
        functools.partial(_res_body, n_tot),
        grid=(B,),
        in_specs=[per_b((N, C)), full((2, C)), full((1, C)), full((1, C)),
                  per_b((N, C))],
        out_specs=per_b((N, C)),
        out_shape=jax.ShapeDtypeStruct((B, N, C), f32),
    )(o_raw, s3, g2r, be2r, xt)

    return jnp.transpose(out, (0, 2, 1)).reshape(B, C, H, W)


# speedup vs baseline: 1.2831x; 1.2831x over previous
import functools

import jax
import jax.numpy as jnp
from jax import lax
from jax.experimental import pallas as pl
from jax.experimental.pallas import tpu as pltpu
from jax.experimental.pallas import tpu_sc as plsc

_EPS = 1e-5
_KNN = 9
_CPAD = 256
_HI = jax.lax.Precision.HIGHEST
_INV_SQRT2 = 0.7071067811865476


def _dot(a, b, dims, precision=_HI):
    return jax.lax.dot_general(a, b, (dims, ((), ())),
                               preferred_element_type=jnp.float32,
                               precision=precision)


def _accum_stats(s_ref, val, is_first):
    st = jnp.concatenate([jnp.sum(val, axis=0, keepdims=True),
                          jnp.sum(val * val, axis=0, keepdims=True)], axis=0)

    @pl.when(is_first)
    def _():
        s_ref[...] = jnp.zeros_like(s_ref)

    s_ref[...] += st


def _affine_from_stats(s_ref, g_ref, be_ref, n_tot):
    inv = 1.0 / n_tot
    mean = s_ref[0:1, :] * inv
    var = s_ref[1:2, :] * inv - mean * mean
    a = g_ref[...] * jax.lax.rsqrt(var + _EPS)
    d = be_ref[...] - mean * a
    return a, d


def _conv1_body(xt_ref, w1_ref, b1_ref, h_ref, s_ref):
    b = pl.program_id(0)
    h = _dot(xt_ref[0], w1_ref[...], ((1,), (1,)), precision=None) + b1_ref[...]
    h_ref[0] = h
    _accum_stats(s_ref, h, b == 0)


def _knn_body(n_tot, h_ref, s1_ref, g1_ref, be1_ref, xr_ref, gidx_ref):
    b = pl.program_id(0)
    a, d = _affine_from_stats(s1_ref, g1_ref, be1_ref, n_tot)
    xr = h_ref[0] * a + d
    xr_ref[0] = jnp.concatenate(
        [xr, jnp.zeros((xr.shape[0], _CPAD - xr.shape[1]), jnp.float32)],
        axis=1)
    nsq = jnp.sum(xr * xr, axis=1, keepdims=True)
    xn = xr * (1.0 / jnp.maximum(jnp.sqrt(nsq), 1e-12))
    n = xn.shape[0]
    sim = _dot(xn, xn, ((1,), (1,)), precision=None)
    sq_row = _dot(jnp.ones((8, xn.shape[1]), jnp.float32), xn * xn,
                  ((1,), (1,)))[0:1]
    v = sq_row - 2.0 * sim
    col = jax.lax.broadcasted_iota(jnp.int32, v.shape, 1)
    idxs = []
    for k in range(_KNN):
        rowmin = jnp.min(v, axis=1, keepdims=True)
        idx = jnp.min(jnp.where(v == rowmin, col, n), axis=1, keepdims=True)
        idxs.append(idx + b * n)
        v = jnp.where(col == idx, jnp.inf, v)
    gidx_ref[0] = jnp.concatenate(idxs, axis=1)


def _zgp_body(xr_ref, w_ref, zgp_ref):
    zgp_ref[0] = _dot(xr_ref[0], w_ref[...], ((1,), (0,)))


def _zg_body(zgp_ref, amax_ref, wdf_ref, bg_ref, zg_ref, s2_ref):
    b = pl.program_id(0)
    zg = zgp_ref[0] + _dot(amax_ref[0], wdf_ref[...], ((1,), (0,))) \
        + bg_ref[...]
    zg_ref[0] = zg
    _accum_stats(s2_ref, zg, b == 0)


_SC_ITEMS = 8
_SC_CHUNKS = 32


def _sc_gather_max(table_ref, idx_ref, out_ref, idxv, rows0, rows1, outv,
                   sem0, sem1):
    wid = lax.axis_index("s") * 2 + lax.axis_index("c")
    base = wid * (_SC_ITEMS * _SC_CHUNKS)
    nidx = _SC_ITEMS * _KNN
    rows = (rows0, rows1)
    sems = (sem0, sem1)
    pltpu.sync_copy(idx_ref.at[pl.ds(base * _KNN, _SC_CHUNKS * nidx)], idxv)
    zero = jnp.zeros((16,), jnp.float32)
    for i in range(_SC_ITEMS):
        for c in range(12, _CPAD // 16):
            outv[i, c * 16:(c + 1) * 16] = zero
    for b in range(2):
        pltpu.async_copy(table_ref.at[idxv.at[pl.ds(b * nidx, nidx)]],
                         rows[b], sems[b])

    def step(g, carry):
        for b in range(2):
            cur = g * 2 + b
            pltpu.make_async_copy(
                table_ref.at[idxv.at[pl.ds(0, nidx)]], rows[b],
                sems[b]).wait()
            r = rows[b]
            for i in range(_SC_ITEMS):
                for c in range(12):
                    sl = slice(c * 16, (c + 1) * 16)
                    j = i * _KNN
                    m01 = jnp.maximum(r[j, sl], r[j + 1, sl])
                    m23 = jnp.maximum(r[j + 2, sl], r[j + 3, sl])
                    m45 = jnp.maximum(r[j + 4, sl], r[j + 5, sl])
                    m67 = jnp.maximum(r[j + 6, sl], r[j + 7, sl])
                    m = jnp.maximum(jnp.maximum(m01, m23),
                                    jnp.maximum(m45, m67))
                    outv[i, sl] = jnp.maximum(m, r[j + 8, sl])
            pltpu.sync_copy(outv,
                            out_ref.at[pl.ds(base + cur * _SC_ITEMS,
                                             _SC_ITEMS)])
            nxt = cur + 2

            @pl.when(nxt < _SC_CHUNKS)
            def _():
                pltpu.async_copy(
                    table_ref.at[idxv.at[pl.ds(nxt * nidx, nidx)]],
                    rows[b], sems[b])
        return carry

    lax.fori_loop(0, _SC_CHUNKS // 2, step, 0)


def _head_body(n_tot, zg_ref, s2_ref, gg_ref, beg_ref, w2_ref, b2_ref,
               o_ref, s3_ref):
    b = pl.program_id(0)
    a, d = _affine_from_stats(s2_ref, gg_ref, beg_ref, n_tot)
    t = zg_ref[0] * a + d
    z = 0.5 * t * (1.0 + jax.lax.erf(t * _INV_SQRT2))
    o = _dot(z, w2_ref[...], ((1,), (1,))) + b2_ref[...]
    o_ref[0] = o
    _accum_stats(s3_ref, o, b == 0)


def _res_body(n_tot, o_ref, s3_ref, g2_ref, be2_ref, xt_ref, out_ref):
    a, d = _affine_from_stats(s3_ref, g2_ref, be2_ref, n_tot)
    out_ref[0] = o_ref[0] * a + d + xt_ref[0]


def _block_diag(wb, c_in, c_out, groups, dtype):
    ig, og = c_in // groups, c_out // groups
    m = jnp.zeros((c_in, c_out), dtype)
    for g in range(groups):
        m = m.at[g * ig:(g + 1) * ig, g * og:(g + 1) * og].set(
            jnp.transpose(wb[g]))
    return m


def kernel(x, W1, b1, g1, be1, Wg, bg, gg, beg, W2, b2, g2, be2):
    B, C, H, W = x.shape
    N = H * W
    Cout = Wg.shape[0]
    groups = 4
    n_tot = float(B * N)
    f32 = jnp.float32

    xt = jnp.transpose(x.reshape(B, C, N), (0, 2, 1))
    wg3 = Wg.reshape(groups, Cout // groups, (2 * C) // groups)
    wxr = _block_diag(wg3[:, :, 0::2], C, Cout, groups, f32)
    wdf = _block_diag(wg3[:, :, 1::2], C, Cout, groups, f32)

    row = lambda v: v.reshape(1, -1).astype(f32)
    b1r, g1r, be1r = row(b1), row(g1), row(be1)
    bgr, ggr, begr = row(bg), row(gg), row(beg)
    b2r, g2r, be2r = row(b2), row(g2), row(be2)

    full = lambda shape: pl.BlockSpec(shape, lambda b: (0,) * len(shape))
    per_b = lambda shape: pl.BlockSpec((1,) + shape,
                                       lambda b: (b,) + (0,) * len(shape))

    h_raw, s1 = pl.pallas_call(
        _conv1_body,
        grid=(B,),
        in_specs=[per_b((N, C)), full((C, C)), full((1, C))],
        out_specs=[per_b((N, C)), full((2, C))],
        out_shape=[jax.ShapeDtypeStruct((B, N, C), f32),
                   jax.ShapeDtypeStruct((2, C), f32)],
    )(xt, W1, b1r)

    xr_all, gidx = pl.pallas_call(
        functools.partial(_knn_body, n_tot),
        grid=(B,),
        in_specs=[per_b((N, C)), full((2, C)), full((1, C)), full((1, C))],
        out_specs=[per_b((N, _CPAD)), per_b((N, _KNN))],
        out_shape=[jax.ShapeDtypeStruct((B, N, _CPAD), f32),
                   jax.ShapeDtypeStruct((B, N, _KNN), jnp.int32)],
    )(h_raw, s1, g1r, be1r)

    sc_mesh = plsc.VectorSubcoreMesh(core_axis_name="c", subcore_axis_name="s")
    amax = pl.kernel(
        _sc_gather_max,
        mesh=sc_mesh,
        out_type=jax.ShapeDtypeStruct((B * N, _CPAD), f32),
        scratch_types=[
            pltpu.VMEM((_SC_ITEMS * _SC_CHUNKS * _KNN,), jnp.int32),
            pltpu.VMEM((_SC_ITEMS * _KNN, _CPAD), f32),
            pltpu.VMEM((_SC_ITEMS * _KNN, _CPAD), f32),
            pltpu.VMEM((_SC_ITEMS, _CPAD), f32),
            pltpu.SemaphoreType.DMA,
            pltpu.SemaphoreType.DMA,
        ],
    )(xr_all.reshape(B * N, _CPAD), gidx.reshape(B * N * _KNN))
    amax = amax.reshape(B, N, _CPAD)

    pad_rows = lambda w: jnp.concatenate(
        [w, jnp.zeros((_CPAD - C, Cout), f32)], axis=0)

    zgp = pl.pallas_call(
        _zgp_body,
        grid=(B,),
        in_specs=[per_b((N, _CPAD)), full((_CPAD, Cout))],
        out_specs=per_b((N, Cout)),
        out_shape=jax.ShapeDtypeStruct((B, N, Cout), f32),
    )(xr_all, pad_rows(wxr - wdf))

    zg_raw, s2 = pl.pallas_call(
        _zg_body,
        grid=(B,),
        in_specs=[per_b((N, Cout)), per_b((N, _CPAD)), full((_CPAD, Cout)),
                  full((1, Cout))],
        out_specs=[per_b((N, Cout)), full((2, Cout))],
        out_shape=[jax.ShapeDtypeStruct((B, N, Cout), f32),
                   jax.ShapeDtypeStruct((2, Cout), f32)],
    )(zgp, amax, pad_rows(wdf), bgr)

    o_raw, s3 = pl.pallas_call(
        functools.partial(_head_body, n_tot),
        grid=(B,),
        in_specs=[per_b((N, Cout)), full((2, Cout)), full((1, Cout)),
                  full((1, Cout)), full((C, Cout)), full((1, C))],
        out_specs=[per_b((N, C)), full((2, C))],
        out_shape=[jax.ShapeDtypeStruct((B, N, C), f32),
                   jax.ShapeDtypeStruct((2, C), f32)],
    )(zg_raw, s2, ggr, begr, W2, b2r)

    out = pl.pallas_call(
        functools.partial(_res_body, n_tot),
        grid=(B,),
        in_specs=[per_b((N, C)), full((2, C)), full((1, C)), full((1, C)),
                  per_b((N, C))],
        out_specs=per_b((N, C)),
        out_shape=jax.ShapeDtypeStruct((B, N, C), f32),
    )(o_raw, s3, g2r, be2r, xt)

    return jnp.transpose(out, (0, 2, 1)).reshape(B, C, H, W)


# speedup vs baseline: 1.4108x; 1.0995x over previous
import functools

import jax
import jax.numpy as jnp
from jax import lax
from jax.experimental import pallas as pl
from jax.experimental.pallas import tpu as pltpu
from jax.experimental.pallas import tpu_sc as plsc

_EPS = 1e-5
_KNN = 9
_CPAD = 256
_HI = jax.lax.Precision.HIGHEST
_INV_SQRT2 = 0.7071067811865476


def _dot(a, b, dims, precision=_HI):
    return jax.lax.dot_general(a, b, (dims, ((), ())),
                               preferred_element_type=jnp.float32,
                               precision=precision)


def _accum_stats(s_ref, val, is_first):
    st = jnp.concatenate([jnp.sum(val, axis=0, keepdims=True),
                          jnp.sum(val * val, axis=0, keepdims=True)], axis=0)

    @pl.when(is_first)
    def _():
        s_ref[...] = jnp.zeros_like(s_ref)

    s_ref[...] += st


def _affine_from_stats(s_ref, g_ref, be_ref, n_tot):
    inv = 1.0 / n_tot
    mean = s_ref[0:1, :] * inv
    var = s_ref[1:2, :] * inv - mean * mean
    a = g_ref[...] * jax.lax.rsqrt(var + _EPS)
    d = be_ref[...] - mean * a
    return a, d


def _conv1_body(xt_ref, w1_ref, b1_ref, h_ref, s_ref):
    b = pl.program_id(0)
    h = _dot(xt_ref[0], w1_ref[...], ((1,), (1,)), precision=None) + b1_ref[...]
    h_ref[0] = h
    _accum_stats(s_ref, h, b == 0)


def _knn_body(n_tot, h_ref, s1_ref, g1_ref, be1_ref, xr_ref, gidx_ref):
    b = pl.program_id(0)
    a, d = _affine_from_stats(s1_ref, g1_ref, be1_ref, n_tot)
    xr = h_ref[0] * a + d
    xr_ref[0] = jnp.concatenate(
        [xr, jnp.zeros((xr.shape[0], _CPAD - xr.shape[1]), jnp.float32)],
        axis=1)
    nsq = jnp.sum(xr * xr, axis=1, keepdims=True)
    xn = xr * (1.0 / jnp.maximum(jnp.sqrt(nsq), 1e-12))
    n = xn.shape[0]
    sim = _dot(xn, xn, ((1,), (1,)), precision=None)
    sq_row = _dot(jnp.ones((8, xn.shape[1]), jnp.float32), xn * xn,
                  ((1,), (1,)))[0:1]
    v = sq_row - 2.0 * sim
    col = jax.lax.broadcasted_iota(jnp.int32, v.shape, 1)
    idxs = []
    for k in range(_KNN):
        rowmin = jnp.min(v, axis=1, keepdims=True)
        idx = jnp.min(jnp.where(v == rowmin, col, n), axis=1, keepdims=True)
        idxs.append(idx + b * n)
        v = jnp.where(col == idx, jnp.inf, v)
    gidx_ref[0] = jnp.concatenate(idxs, axis=1)


def _zgp_body(xr_ref, w_ref, zgp_ref):
    zgp_ref[0] = _dot(xr_ref[0], w_ref[...], ((1,), (0,)))


def _zg_body(zgp_ref, amax_ref, wdf_ref, bg_ref, zg_ref, s2_ref):
    b = pl.program_id(0)
    zg = zgp_ref[0] + _dot(amax_ref[0], wdf_ref[...], ((1,), (0,))) \
        + bg_ref[...]
    zg_ref[0] = zg
    _accum_stats(s2_ref, zg, b == 0)


_SC_ITEMS = 8
_SC_CHUNKS = 32


def _sc_gather_max(chunks, table_ref, idx_ref, out_ref, idxv, rows0, rows1,
                   outv, sem0, sem1):
    wid = lax.axis_index("s") * 2 + lax.axis_index("c")
    base = wid * (_SC_ITEMS * chunks)
    nidx = _SC_ITEMS * _KNN
    rows = (rows0, rows1)
    sems = (sem0, sem1)
    pltpu.sync_copy(idx_ref.at[pl.ds(base * _KNN, chunks * nidx)], idxv)
    zero = jnp.zeros((16,), jnp.float32)
    for i in range(_SC_ITEMS):
        for c in range(12, _CPAD // 16):
            outv[i, c * 16:(c + 1) * 16] = zero
    for b in range(2):
        pltpu.async_copy(table_ref.at[idxv.at[pl.ds(b * nidx, nidx)]],
                         rows[b], sems[b])

    def step(g, carry):
        for b in range(2):
            cur = g * 2 + b
            pltpu.make_async_copy(
                table_ref.at[idxv.at[pl.ds(0, nidx)]], rows[b],
                sems[b]).wait()
            r = rows[b]
            for i in range(_SC_ITEMS):
                for c in range(12):
                    sl = slice(c * 16, (c + 1) * 16)
                    j = i * _KNN
                    m01 = jnp.maximum(r[j, sl], r[j + 1, sl])
                    m23 = jnp.maximum(r[j + 2, sl], r[j + 3, sl])
                    m45 = jnp.maximum(r[j + 4, sl], r[j + 5, sl])
                    m67 = jnp.maximum(r[j + 6, sl], r[j + 7, sl])
                    m = jnp.maximum(jnp.maximum(m01, m23),
                                    jnp.maximum(m45, m67))
                    outv[i, sl] = jnp.maximum(m, r[j + 8, sl])
            pltpu.sync_copy(outv,
                            out_ref.at[pl.ds(base + cur * _SC_ITEMS,
                                             _SC_ITEMS)])
            nxt = cur + 2

            @pl.when(nxt < chunks)
            def _():
                pltpu.async_copy(
                    table_ref.at[idxv.at[pl.ds(nxt * nidx, nidx)]],
                    rows[b], sems[b])
        return carry

    lax.fori_loop(0, chunks // 2, step, 0)


def _head_body(n_tot, zg_ref, s2_ref, gg_ref, beg_ref, w2_ref, b2_ref,
               o_ref, s3_ref):
    b = pl.program_id(0)
    a, d = _affine_from_stats(s2_ref, gg_ref, beg_ref, n_tot)
    t = zg_ref[0] * a + d
    z = 0.5 * t * (1.0 + jax.lax.erf(t * _INV_SQRT2))
    o = _dot(z, w2_ref[...], ((1,), (1,))) + b2_ref[...]
    o_ref[0] = o
    _accum_stats(s3_ref, o, b == 0)


def _res_body(n_tot, o_ref, s3_ref, g2_ref, be2_ref, xt_ref, out_ref):
    a, d = _affine_from_stats(s3_ref, g2_ref, be2_ref, n_tot)
    out_ref[0] = o_ref[0] * a + d + xt_ref[0]


def _block_diag(wb, c_in, c_out, groups, dtype):
    ig, og = c_in // groups, c_out // groups
    m = jnp.zeros((c_in, c_out), dtype)
    for g in range(groups):
        m = m.at[g * ig:(g + 1) * ig, g * og:(g + 1) * og].set(
            jnp.transpose(wb[g]))
    return m


def kernel(x, W1, b1, g1, be1, Wg, bg, gg, beg, W2, b2, g2, be2):
    B, C, H, W = x.shape
    N = H * W
    Cout = Wg.shape[0]
    groups = 4
    n_tot = float(B * N)
    f32 = jnp.float32

    xt = jnp.transpose(x.reshape(B, C, N), (0, 2, 1))
    wg3 = Wg.reshape(groups, Cout // groups, (2 * C) // groups)
    wxr = _block_diag(wg3[:, :, 0::2], C, Cout, groups, f32)
    wdf = _block_diag(wg3[:, :, 1::2], C, Cout, groups, f32)

    row = lambda v: v.reshape(1, -1).astype(f32)
    b1r, g1r, be1r = row(b1), row(g1), row(be1)
    bgr, ggr, begr = row(bg), row(gg), row(beg)
    b2r, g2r, be2r = row(b2), row(g2), row(be2)

    full = lambda shape: pl.BlockSpec(shape, lambda b: (0,) * len(shape))
    per_b = lambda shape: pl.BlockSpec((1,) + shape,
                                       lambda b: (b,) + (0,) * len(shape))

    h_raw, s1 = pl.pallas_call(
        _conv1_body,
        grid=(B,),
        in_specs=[per_b((N, C)), full((C, C)), full((1, C))],
        out_specs=[per_b((N, C)), full((2, C))],
        out_shape=[jax.ShapeDtypeStruct((B, N, C), f32),
                   jax.ShapeDtypeStruct((2, C), f32)],
    )(xt, W1, b1r)

    GB = 2
    chunks = GB * N // (32 * _SC_ITEMS)
    sc_mesh = plsc.VectorSubcoreMesh(core_axis_name="c", subcore_axis_name="s")
    xr_parts, amax_parts = [], []
    for g in range(B // GB):
        xr_g, gidx_g = pl.pallas_call(
            functools.partial(_knn_body, n_tot),
            grid=(GB,),
            in_specs=[pl.BlockSpec((1, N, C), lambda i, g=g: (g * GB + i, 0, 0)),
                      full((2, C)), full((1, C)), full((1, C))],
            out_specs=[per_b((N, _CPAD)), per_b((N, _KNN))],
            out_shape=[jax.ShapeDtypeStruct((GB, N, _CPAD), f32),
                       jax.ShapeDtypeStruct((GB, N, _KNN), jnp.int32)],
        )(h_raw, s1, g1r, be1r)
        amax_g = pl.kernel(
            functools.partial(_sc_gather_max, chunks),
            mesh=sc_mesh,
            out_type=jax.ShapeDtypeStruct((GB * N, _CPAD), f32),
            scratch_types=[
                pltpu.VMEM((chunks * _SC_ITEMS * _KNN,), jnp.int32),
                pltpu.VMEM((_SC_ITEMS * _KNN, _CPAD), f32),
                pltpu.VMEM((_SC_ITEMS * _KNN, _CPAD), f32),
                pltpu.VMEM((_SC_ITEMS, _CPAD), f32),
                pltpu.SemaphoreType.DMA,
                pltpu.SemaphoreType.DMA,
            ],
        )(xr_g.reshape(GB * N, _CPAD), gidx_g.reshape(GB * N * _KNN))
        xr_parts.append(xr_g)
        amax_parts.append(amax_g.reshape(GB, N, _CPAD))
    xr_all = jnp.concatenate(xr_parts, axis=0)
    amax = jnp.concatenate(amax_parts, axis=0)

    pad_rows = lambda w: jnp.concatenate(
        [w, jnp.zeros((_CPAD - C, Cout), f32)], axis=0)

    zgp = pl.pallas_call(
        _zgp_body,
        grid=(B,),
        in_specs=[per_b((N, _CPAD)), full((_CPAD, Cout))],
        out_specs=per_b((N, Cout)),
        out_shape=jax.ShapeDtypeStruct((B, N, Cout), f32),
    )(xr_all, pad_rows(wxr - wdf))

    zg_raw, s2 = pl.pallas_call(
        _zg_body,
        grid=(B,),
        in_specs=[per_b((N, Cout)), per_b((N, _CPAD)), full((_CPAD, Cout)),
                  full((1, Cout))],
        out_specs=[per_b((N, Cout)), full((2, Cout))],
        out_shape=[jax.ShapeDtypeStruct((B, N, Cout), f32),
                   jax.ShapeDtypeStruct((2, Cout), f32)],
    )(zgp, amax, pad_rows(wdf), bgr)

    o_raw, s3 = pl.pallas_call(
        functools.partial(_head_body, n_tot),
        grid=(B,),
        in_specs=[per_b((N, Cout)), full((2, Cout)), full((1, Cout)),
                  full((1, Cout)), full((C, Cout)), full((1, C))],
        out_specs=[per_b((N, C)), full((2, C))],
        out_shape=[jax.ShapeDtypeStruct((B, N, C), f32),
                   jax.ShapeDtypeStruct((2, C), f32)],
    )(zg_raw, s2, ggr, begr, W2, b2r)

    out = pl.pallas_call(
        functools.partial(_res_body, n_tot),
        grid=(B,),
        in_specs=[per_b((N, C)), full((2, C)), full((1, C)), full((1, C)),
                  per_b((N, C))],
        out_specs=per_b((N, C)),
        out_shape=jax.ShapeDtypeStruct((B, N, C), f32),
    )(o_raw, s3, g2r, be2r, xt)

    return jnp.transpose(out, (0, 2, 1)).reshape(B, C, H, W)
